# baseline (speedup 1.0000x reference)
import functools

import jax
import jax.numpy as jnp
from jax import lax
from jax.experimental import pallas as pl
from jax.experimental.pallas import tpu as pltpu
from jax.experimental.pallas import tpu_sc as plsc

D = 128


def _lookup_body(x_hbm, tab_hbm, out_hbm, idx_s):
    pltpu.sync_copy(x_hbm, idx_s)
    i = idx_s[0]
    pltpu.sync_copy(tab_hbm.at[pl.ds(i, 1)], out_hbm)


@jax.jit
def kernel(x, table):
    idx = jnp.reshape(x, (1,)).astype(jnp.int32)
    mesh = plsc.ScalarSubcoreMesh(axis_name="c", num_cores=1)
    run = functools.partial(
        pl.kernel,
        mesh=mesh,
        out_type=jax.ShapeDtypeStruct((1, D), jnp.float32),
        scratch_types=[
            pltpu.SMEM((1,), jnp.int32),
        ],
        compiler_params=pltpu.CompilerParams(
            disable_bounds_checks=True,
            disable_semaphore_checks=True,
            skip_device_barrier=True,
        ),
    )(_lookup_body)
    return run(idx, table)


# speedup vs baseline: 1.0048x; 1.0048x over previous
import functools

import jax
import jax.numpy as jnp
from jax import lax
from jax.experimental import pallas as pl
from jax.experimental.pallas import tpu as pltpu
from jax.experimental.pallas import tpu_sc as plsc

D = 128


def _lookup_body(x_hbm, tab_hbm, out_hbm, idx_s):
    pltpu.sync_copy(x_hbm, idx_s)
    i = idx_s[0]
    pltpu.sync_copy(tab_hbm.at[pl.ds(i, 1)], out_hbm)


@jax.jit
def kernel(x, table):
    idx = jnp.reshape(x, (1,)).astype(jnp.int32)
    mesh = plsc.ScalarSubcoreMesh(axis_name="c", num_cores=1)
    run = functools.partial(
        pl.kernel,
        mesh=mesh,
        out_type=jax.ShapeDtypeStruct((1, D), jnp.float32),
        scratch_types=[
            pltpu.SMEM((1,), jnp.int32),
        ],
    )(_lookup_body)
    return run(idx, table)
